# baseline (speedup 1.0000x reference)
import functools

import jax
import jax.numpy as jnp
from jax import lax
from jax.experimental import pallas as pl
from jax.experimental.pallas import tpu as pltpu
from jax.experimental.pallas import tpu_sc as plsc

NC, NS, LANES = 2, 16, 16
NW = NC * NS
B = 4096
DIM = 64
NEG = 20
BPW = B // NW
HPW = BPW // 2
VP = 500000
NBUF = 3

_MESH = plsc.VectorSubcoreMesh(core_axis_name="c", subcore_axis_name="s")
_PARAMS = pltpu.CompilerParams(use_tc_tiling_on_sc=False)


def _select_half(dst, src, pv, init, k):
    m0 = pv[pl.ds((2 * k) * LANES, LANES)] != 0
    m1 = pv[pl.ds((2 * k + 1) * LANES, LANES)] != 0
    for c0 in range(0, DIM, LANES):
        v0 = jnp.where(m0, src[2 * k, pl.ds(DIM + c0, LANES)],
                       src[2 * k, pl.ds(c0, LANES)])
        v1 = jnp.where(m1, src[2 * k + 1, pl.ds(DIM + c0, LANES)],
                       src[2 * k + 1, pl.ds(c0, LANES)])
        if init:
            dst[k, pl.ds(c0, LANES)] = v0
            dst[k, pl.ds(DIM + c0, LANES)] = v1
        else:
            plsc.addupdate(dst.at[k, pl.ds(c0, LANES)], v0)
            plsc.addupdate(dst.at[k, pl.ds(DIM + c0, LANES)], v1)


def _sc_center(Wv2, cidx, cpar):
    @functools.partial(
        pl.kernel,
        out_type=jax.ShapeDtypeStruct((B // 2, 2 * DIM), jnp.float32),
        mesh=_MESH,
        compiler_params=_PARAMS,
        scratch_types=[
            pltpu.VMEM((BPW,), jnp.int32),
            pltpu.VMEM((BPW * LANES,), jnp.int32),
            pltpu.VMEM((BPW, 2 * DIM), jnp.float32),
            pltpu.VMEM((HPW, 2 * DIM), jnp.float32),
        ],
    )
    def k(wv_hbm, c_hbm, p_hbm, oc_hbm, civ, cpv, cbuf, obuf):
        wid = lax.axis_index("c") * NS + lax.axis_index("s")
        base = wid * BPW
        pltpu.sync_copy(c_hbm.at[pl.ds(base, BPW)], civ)
        pltpu.sync_copy(p_hbm.at[pl.ds(base * LANES, BPW * LANES)], cpv)
        pltpu.sync_copy(wv_hbm.at[civ], cbuf)

        @pl.loop(0, HPW)
        def _(i):
            _select_half(obuf, cbuf, cpv, True, i)

        pltpu.sync_copy(obuf, oc_hbm.at[pl.ds(wid * HPW, HPW)])

    return k(Wv2, cidx, cpar)


def _sc_target_neg(Wu2, tidx, tpar, nidx, npar):
    out_t = [jax.ShapeDtypeStruct((B // 2, 2 * DIM), jnp.float32)] * 2

    @functools.partial(
        pl.kernel,
        out_type=out_t,
        mesh=_MESH,
        compiler_params=_PARAMS,
        scratch_types=[
            pltpu.VMEM((BPW,), jnp.int32),
            pltpu.VMEM((BPW * LANES,), jnp.int32),
            pltpu.VMEM((NEG, BPW), jnp.int32),
            pltpu.VMEM((BPW, 2 * DIM), jnp.float32),
            pltpu.VMEM((HPW, 2 * DIM), jnp.float32),
            pltpu.VMEM((HPW, 2 * DIM), jnp.float32),
        ]
        + [pltpu.VMEM((BPW, 2 * DIM), jnp.float32)] * NBUF
        + [pltpu.VMEM((BPW * LANES,), jnp.int32)] * NBUF
        + [pltpu.SemaphoreType.DMA] * (2 * NBUF + 1),
    )
    def k(wu_hbm, t_hbm, tp_hbm, n_hbm, np_hbm, ot_hbm, on_hbm,
          tiv, tpv, niv, tbuf, obuf, acc,
          nb0, nb1, nb2, pb0, pb1, pb2,
          s0, s1, s2, q0, q1, q2, st):
        sid = lax.axis_index("s")
        wid = lax.axis_index("c") * NS + sid
        base = wid * BPW

        pltpu.sync_copy(t_hbm.at[pl.ds(base, BPW)], tiv)
        pltpu.sync_copy(tp_hbm.at[pl.ds(base * LANES, BPW * LANES)], tpv)
        pltpu.sync_copy(n_hbm.at[wid], niv)

        ft = pltpu.async_copy(wu_hbm.at[tiv], tbuf, st)

        nbufs = [nb0, nb1, nb2]
        pbufs = [pb0, pb1, pb2]
        sems = [s0, s1, s2]
        qems = [q0, q1, q2]
        pend = [
            (pltpu.async_copy(wu_hbm.at[niv.at[j]], nbufs[j], sems[j]),
             pltpu.async_copy(np_hbm.at[wid, j], pbufs[j], qems[j]))
            for j in range(NBUF)
        ]
        for j in range(NEG):
            b = j % NBUF
            pend[b][0].wait()
            pend[b][1].wait()
            buf = nbufs[b]
            pv = pbufs[b]
            init = j == 0

            @pl.loop(0, HPW)
            def _(i, buf=buf, pv=pv, init=init):
                _select_half(acc, buf, pv, init, i)

            nxt = j + NBUF
            if nxt < NEG:
                pend[b] = (
                    pltpu.async_copy(wu_hbm.at[niv.at[nxt]], nbufs[b],
                                     sems[b]),
                    pltpu.async_copy(np_hbm.at[wid, nxt], pbufs[b], qems[b]),
                )

        ft.wait()

        @pl.loop(0, HPW)
        def _(i):
            _select_half(obuf, tbuf, tpv, True, i)

        pltpu.sync_copy(obuf, ot_hbm.at[pl.ds(wid * HPW, HPW)])
        pltpu.sync_copy(acc, on_hbm.at[pl.ds(wid * HPW, HPW)])

    return k(Wu2, tidx, tpar, nidx, npar)


def _tc_loss(ce, te, ns):
    def body(c_ref, t_ref, n_ref, o_ref):
        c = c_ref[...]
        t = t_ref[...]
        n = n_ref[...]
        sp = c * t
        sn = c * n
        pos_lo = jnp.sum(sp[:, :DIM], axis=1)
        pos_hi = jnp.sum(sp[:, DIM:], axis=1)
        neg_lo = -jnp.sum(sn[:, :DIM], axis=1)
        neg_hi = -jnp.sum(sn[:, DIM:], axis=1)

        def logsig(x):
            return jnp.minimum(x, 0.0) - jnp.log1p(jnp.exp(-jnp.abs(x)))

        tot = (jnp.sum(logsig(pos_lo)) + jnp.sum(logsig(pos_hi))
               + jnp.sum(logsig(neg_lo)) + jnp.sum(logsig(neg_hi)))
        o_ref[...] = jnp.reshape(-tot / B, (1, 1))

    return pl.pallas_call(
        body,
        out_shape=jax.ShapeDtypeStruct((1, 1), jnp.float32),
    )(ce, te, ns)


def _splat16(x):
    return jnp.repeat((x & 1).astype(jnp.int32), LANES)


def kernel(center_words, target_words, negative_words, Wv, Wu):
    Wv2 = Wv.reshape(VP, 2 * DIM)
    Wu2 = Wu.reshape(VP, 2 * DIM)
    ci = (center_words >> 1).astype(jnp.int32)
    ti = (target_words >> 1).astype(jnp.int32)
    cp = _splat16(center_words)
    tp = _splat16(target_words)
    nidx = jnp.transpose(negative_words.reshape(NW, BPW, NEG), (0, 2, 1))
    ni = (nidx >> 1).astype(jnp.int32)
    npar = _splat16(nidx.reshape(-1)).reshape(NW, NEG, BPW * LANES)
    te, nsum = _sc_target_neg(Wu2, ti, tp, ni, npar)
    ce = _sc_center(Wv2, ci, cp)
    out = _tc_loss(ce, te, nsum)
    return jnp.reshape(out, ())
